# baseline (speedup 1.0000x reference)
import functools

import jax
import jax.numpy as jnp
from jax import lax
from jax.experimental import pallas as pl
from jax.experimental.pallas import tpu as pltpu
from jax.experimental.pallas import tpu_sc as plsc

N_TOK = 16384
BATCH = 8
OUT_LEN = 4096
D_FEAT = 256
PTS_OUT = 3

T = 256
W = 248
R = 256
KMAX = 5
TILES_PER_STEP = 4

_SC_WORKERS = 16
_SC_CHUNK = N_TOK // _SC_WORKERS
_LANES = 16
_JW = OUT_LEN // _SC_WORKERS


def _sc_points_body(batch_hbm, pts_hbm, out_hbm,
                    slice_v, cnt_v, meta_v, pts_v, stage_v, shared, all_v):
    cid = lax.axis_index("c")
    sid = lax.axis_index("s")
    lane = lax.iota(jnp.int32, _LANES)

    @pl.when(cid == 0)
    def _count():
        pltpu.sync_copy(pts_hbm, pts_v)
        pltpu.sync_copy(batch_hbm.at[pl.ds(sid * _SC_CHUNK, _SC_CHUNK)], slice_v)
        counts = jnp.zeros((_LANES,), jnp.int32)
        for b in range(BATCH):
            def step(i, vacc, b=b):
                v = slice_v[pl.ds(i * _LANES, _LANES)]
                return vacc + (v == b).astype(jnp.int32)
            vacc = lax.fori_loop(0, _SC_CHUNK // _LANES, step,
                                 jnp.zeros((_LANES,), jnp.int32))
            counts = counts + jnp.where(lane == b, jnp.sum(vacc), 0)
        cnt_v[...] = counts
        pltpu.sync_copy(cnt_v, shared.at[pl.ds(sid * _LANES, _LANES)])

    plsc.subcore_barrier()

    @pl.when((cid == 0) & (sid == 0))
    def _reduce():
        pltpu.sync_copy(shared, all_v)
        total = jnp.zeros((_LANES,), jnp.int32)
        for w in range(_SC_WORKERS):
            total = total + all_v[pl.ds(w * _LANES, _LANES)]
        meta_v[pl.ds(0, _LANES)] = total
        meta_v[pl.ds(_LANES, _LANES)] = plsc.cumsum(total) - total
        pltpu.sync_copy(meta_v, shared.at[pl.ds(0, 2 * _LANES)])

    plsc.subcore_barrier()

    @pl.when(cid == 0)
    def _points():
        pltpu.sync_copy(shared.at[pl.ds(0, 2 * _LANES)], meta_v)
        cv = meta_v[pl.ds(0, _LANES)]
        sv = meta_v[pl.ds(_LANES, _LANES)]
        j_base = sid * _JW
        for b in range(BATCH):
            n = jnp.sum(jnp.where(lane == b, cv, 0))
            s = jnp.sum(jnp.where(lane == b, sv, 0))
            nm1 = n - 1

            def group(g, carry, b=b, n=n, s=s, nm1=nm1):
                js = j_base + g * _LANES + lane
                src = jnp.minimum((js * n) >> 12, nm1)
                gidx = s + src
                gidx = jnp.where(gidx < 0, gidx + N_TOK, gidx)
                idx4 = gidx * 4
                for c in range(PTS_OUT):
                    val = plsc.load_gather(pts_v, [idx4 + c])
                    stage_v[pl.ds((b * PTS_OUT + c) * _JW + g * _LANES,
                                  _LANES)] = val
                return carry

            lax.fori_loop(0, _JW // _LANES, group, jnp.int32(0))
        for b in range(BATCH):
            for c in range(PTS_OUT):
                row = b * PTS_OUT + c
                pltpu.sync_copy(
                    stage_v.at[pl.ds(row * _JW, _JW)],
                    out_hbm.at[pl.ds(row * OUT_LEN + j_base, _JW)])


def _sc_points(batch32, pts_flat):
    mesh = plsc.VectorSubcoreMesh(core_axis_name="c", subcore_axis_name="s")
    f = functools.partial(
        pl.kernel,
        out_type=jax.ShapeDtypeStruct((BATCH * PTS_OUT * OUT_LEN,),
                                      jnp.float32),
        mesh=mesh,
        compiler_params=pltpu.CompilerParams(needs_layout_passes=False),
        scratch_types=[
            pltpu.VMEM((_SC_CHUNK,), jnp.int32),
            pltpu.VMEM((_LANES,), jnp.int32),
            pltpu.VMEM((2 * _LANES,), jnp.int32),
            pltpu.VMEM((N_TOK * 4,), jnp.float32),
            pltpu.VMEM((BATCH * PTS_OUT * _JW,), jnp.float32),
            pltpu.VMEM_SHARED((_SC_WORKERS * _LANES,), jnp.int32),
            pltpu.VMEM((_SC_WORKERS * _LANES,), jnp.int32),
        ],
    )(_sc_points_body)
    return f(batch32, pts_flat)


def _gather_body(batch_ref, feat_ref, out_f_ref):
    b = pl.program_id(0)
    t = pl.program_id(1)
    bv = batch_ref[...]
    n = jnp.sum((bv == b).astype(jnp.int32))
    s = jnp.sum((bv < b).astype(jnp.int32))
    nm1 = n - 1

    for u in range(TILES_PER_STEP):
        j0 = (t * TILES_PER_STEP + u) * T

        src0 = jnp.minimum((j0 * n) >> 12, nm1)
        srcl = jnp.minimum(((j0 + T - 1) * n) >> 12, nm1)
        lo = s + src0
        hi = s + srcl
        lo = jnp.where(lo < 0, lo + N_TOK, lo)
        hi = jnp.where(hi < 0, hi + N_TOK, hi)

        js = j0 + lax.broadcasted_iota(jnp.int32, (1, T), 1)
        srcv = jnp.minimum((js * n) >> 12, nm1)
        gv = s + srcv
        gv = jnp.where(gv < 0, gv + N_TOK, gv)

        def window(k, lo=lo, gv=gv):
            w0 = lo + k * W
            base = jnp.minimum(w0, N_TOK - R)
            base8 = pl.multiple_of((base >> 3) << 3, 8)
            member = (gv >= w0) & (gv < w0 + W)
            loc = gv - base8
            iota_r = lax.broadcasted_iota(jnp.int32, (R, T), 0)
            sel = jnp.where((iota_r == loc) & member,
                            1.0, 0.0).astype(jnp.bfloat16)
            chunk = feat_ref[pl.ds(base8, R), :].astype(jnp.bfloat16)
            return lax.dot_general(chunk, sel, (((0,), (0,)), ((), ())),
                                   preferred_element_type=jnp.float32)

        col = pl.ds(u * T, T)
        out_f_ref[0, :, col] = window(0)
        for k in range(1, KMAX):
            @pl.when(lo + k * W <= hi)
            def _extra(k=k, lo=lo, gv=gv, col=col):
                out_f_ref[0, :, col] += window(k, lo, gv)


def _gather_transpose(batch2d, feat, interpret=False):
    tstep = T * TILES_PER_STEP
    return pl.pallas_call(
        _gather_body,
        grid=(BATCH, OUT_LEN // tstep),
        in_specs=[
            pl.BlockSpec((N_TOK // 128, 128), lambda i, j: (0, 0)),
            pl.BlockSpec((N_TOK, D_FEAT), lambda i, j: (0, 0)),
        ],
        out_specs=pl.BlockSpec((1, D_FEAT, tstep), lambda i, j: (i, 0, j)),
        out_shape=jax.ShapeDtypeStruct((BATCH, D_FEAT, OUT_LEN), jnp.float32),
        interpret=interpret,
    )(batch2d, feat)


def kernel(points_x, point_features, batch):
    batch32 = batch.astype(jnp.int32)
    batch2d = batch32.reshape(N_TOK // 128, 128)
    out_p = _sc_points(batch32, points_x.reshape(-1))
    out_f = _gather_transpose(batch2d, point_features)
    return (out_p.reshape(BATCH, PTS_OUT, OUT_LEN), out_f)
